# baseline (speedup 1.0000x reference)
import functools

import jax
import jax.numpy as jnp
import numpy as np
from jax import lax
from jax.experimental import pallas as pl
from jax.experimental.pallas import tpu as pltpu
from jax.experimental.pallas import tpu_sc as plsc

N = 10000
E = 160000
C = 16
H = 16
P = 8
EP = E // P
NC, NS = 2, 16
NW = NC * NS
EPW = E // NW
NPT = 640
RPT = N // NS
ONES_PAD = ((EPW + 15) // 16) * 16


@functools.cache
def _mesh():
  return plsc.VectorSubcoreMesh(
      core_axis_name="c", subcore_axis_name="s", num_cores=NC, num_subcores=NS)


TCH = 1000


def _sc_gather_count(ei_hbm, nf_hbm, eft_hbm, xdst_hbm, cnt_hbm, efp_hbm,
                     idx_v, rows_v, ones_v, zbuf_v, efbuf_v, outbuf_v,
                     cnt_sh, sem):
  cid = lax.axis_index("c")
  sid = lax.axis_index("s")
  wid = sid * NC + cid
  base = wid * EPW
  pltpu.sync_copy(ei_hbm.at[0, pl.ds(base, EPW)], idx_v)
  pltpu.async_copy(nf_hbm.at[idx_v], rows_v, sem).wait()
  pltpu.sync_copy(rows_v, xdst_hbm.at[wid])

  stride16 = lax.iota(jnp.int32, 16) * TCH
  for chunk in range(EPW // TCH):
    for a in range(H):
      pltpu.sync_copy(eft_hbm.at[a, pl.ds(base + chunk * TCH, TCH)],
                      efbuf_v.at[pl.ds(a * TCH, TCH)])

    def trans(e, carry):
      for u in range(4):
        row = plsc.load_gather(efbuf_v, [stride16 + (e * 4 + u)])
        outbuf_v[pl.ds((e * 4 + u) * H, H)] = row
      return carry
    lax.fori_loop(0, TCH // 4, trans, 0)
    pltpu.sync_copy(outbuf_v, efp_hbm.at[wid, pl.ds(chunk * TCH * H, TCH * H)])

  def zfill(i, carry):
    zbuf_v[pl.ds(i * 16, 16)] = jnp.zeros((16,), jnp.float32)
    return carry
  lax.fori_loop(0, NPT // 16, zfill, 0)
  pltpu.sync_copy(zbuf_v, cnt_sh.at[pl.ds(sid * NPT, NPT)])

  def ofill(i, carry):
    ones_v[pl.ds(i * 16, 16)] = jnp.ones((16,), jnp.float32)
    return carry
  lax.fori_loop(0, ONES_PAD // 16, ofill, 0)
  plsc.subcore_barrier()

  pltpu.sync_copy(ei_hbm.at[1, pl.ds(base, EPW)], idx_v)
  pltpu.sync_copy(ones_v.at[pl.ds(0, EPW)], cnt_sh.at[idx_v], add=True)
  plsc.subcore_barrier()
  pltpu.sync_copy(cnt_sh.at[pl.ds(sid * NPT, NPT)],
                  cnt_hbm.at[cid, pl.ds(sid * NPT, NPT)])


@functools.cache
def _k1():
  return pl.kernel(
      _sc_gather_count,
      out_type=[jax.ShapeDtypeStruct((NW, EPW, C), jnp.float32),
                jax.ShapeDtypeStruct((NC, NS * NPT), jnp.float32),
                jax.ShapeDtypeStruct((NW, EPW * H), jnp.float32)],
      mesh=_mesh(),
      scratch_types=[pltpu.VMEM((EPW,), jnp.int32),
                     pltpu.VMEM((EPW, C), jnp.float32),
                     pltpu.VMEM((ONES_PAD,), jnp.float32),
                     pltpu.VMEM((NPT,), jnp.float32),
                     pltpu.VMEM((H * TCH,), jnp.float32),
                     pltpu.VMEM((TCH * H,), jnp.float32),
                     pltpu.VMEM_SHARED((NS * NPT,), jnp.float32),
                     pltpu.SemaphoreType.DMA],
      compiler_params=pltpu.CompilerParams(use_tc_tiling_on_sc=False,
                                           needs_layout_passes=False),
  )


def _sc_scatter(tp_hbm, ei_hbm, seg_hbm, idx_v, rows_v, zrows_v, acc_sh):
  cid = lax.axis_index("c")
  sid = lax.axis_index("s")
  wid = sid * NC + cid

  def zfill(i, carry):
    zrows_v[i, :] = jnp.zeros((16,), jnp.float32)
    return carry
  lax.fori_loop(0, RPT, zfill, 0)
  pltpu.sync_copy(zrows_v, acc_sh.at[pl.ds(sid * RPT, RPT)])
  plsc.subcore_barrier()

  pltpu.sync_copy(ei_hbm.at[1, pl.ds(wid * EPW, EPW)], idx_v)
  pltpu.sync_copy(tp_hbm.at[wid], rows_v)
  pltpu.sync_copy(rows_v, acc_sh.at[idx_v], add=True)
  plsc.subcore_barrier()
  pltpu.sync_copy(acc_sh.at[pl.ds(sid * RPT, RPT)], seg_hbm.at[cid, sid])


@functools.cache
def _k3():
  return pl.kernel(
      _sc_scatter,
      out_type=[jax.ShapeDtypeStruct((NC, NS, RPT, C), jnp.float32)],
      mesh=_mesh(),
      scratch_types=[pltpu.VMEM((EPW,), jnp.int32),
                     pltpu.VMEM((EPW, C), jnp.float32),
                     pltpu.VMEM((RPT, C), jnp.float32),
                     pltpu.VMEM_SHARED((N, C), jnp.float32)],
      compiler_params=pltpu.CompilerParams(use_tc_tiling_on_sc=False),
  )


BLK8 = 1000


def _tc_edge(ef_ref, xd_ref, sh_ref, w1_ref, b1_ref, w2_ref, u_ref,
             r_ref, r8_ref, tp_ref):
  f32 = jnp.float32
  bf = jnp.bfloat16
  xb = xd_ref[...].astype(bf)
  hp = jnp.dot(ef_ref[...].astype(bf), w1_ref[...],
               preferred_element_type=f32) + b1_ref[...]
  h = jnp.maximum(hp, 0.0).astype(bf)
  w = jnp.dot(h, w2_ref[...], preferred_element_type=f32)
  xe = jnp.dot(xb, r_ref[...], preferred_element_type=f32)
  z = w * xe
  parts = [z[:, 128 * i:128 * (i + 1)] for i in range(C)]
  while len(parts) > 1:
    parts = [parts[j] + parts[j + 1] for j in range(0, len(parts), 2)]
  tp = parts[0] + jnp.dot(xb, u_ref[...], preferred_element_type=f32)
  tp_ref[...] = tp * (sh_ref[...] @ r8_ref[...])


def _tc_edge_call(ef, xd, sh, w1bd, b1t, w2bd, ubd, rbd, r8):
  grid = EP // BLK8
  full = lambda shape: pl.BlockSpec(shape, lambda i: (0, 0))
  return pl.pallas_call(
      _tc_edge,
      grid=(grid,),
      in_specs=[
          pl.BlockSpec((BLK8, P * H), lambda i: (i, 0)),
          pl.BlockSpec((BLK8, P * C), lambda i: (i, 0)),
          pl.BlockSpec((BLK8, P), lambda i: (i, 0)),
          full((P * H, P * 2 * H)),
          full((1, P * 2 * H)),
          full((P * 2 * H, C * P * C)),
          full((P * C, P * C)),
          full((P * C, C * P * C)),
          full((P, P * C)),
      ],
      out_specs=pl.BlockSpec((BLK8, P * C), lambda i: (i, 0)),
      out_shape=jax.ShapeDtypeStruct((EP, P * C), jnp.float32),
  )(ef, xd, sh, w1bd, b1t, w2bd, ubd, rbd, r8)


NP8 = N // P


def _tc_final(segp_ref, cnt8_ref, nf_ref, wbd_ref, r8_ref, ms_ref,
              bnw_ref, bnb_ref, out_ref):
  seg = segp_ref[0] + segp_ref[1]
  cnt16 = jnp.maximum(cnt8_ref[...] @ r8_ref[...], 1.0)
  pre = seg / cnt16 + nf_ref[...] @ wbd_ref[...]
  m = jnp.mean(pre, axis=0, keepdims=True)
  msq = jnp.mean(pre * pre, axis=0, keepdims=True)
  mu = m @ ms_ref[...]
  var = msq @ ms_ref[...] - mu * mu
  out_ref[...] = (pre - mu) * lax.rsqrt(var + 1e-5) * bnw_ref[...] + bnb_ref[...]


def _tc_final_call(segp, cnt8, nf_p, wbd, r8, ms, bnw_p, bnb_p):
  full = lambda shape: pl.BlockSpec(shape, lambda: tuple(0 for _ in shape))
  return pl.pallas_call(
      _tc_final,
      in_specs=[
          full((NC, NP8, P * C)),
          full((NP8, P)),
          full((NP8, P * C)),
          full((P * C, P * C)),
          full((P, P * C)),
          full((P * C, P * C)),
          full((1, P * C)),
          full((1, P * C)),
      ],
      out_specs=full((NP8, P * C)),
      out_shape=jax.ShapeDtypeStruct((NP8, P * C), jnp.float32),
  )(segp, cnt8, nf_p, wbd, r8, ms, bnw_p, bnb_p)


def _np_consts():
  eye8 = np.eye(P, dtype=np.float32)
  eye16 = np.eye(C, dtype=np.float32)
  ones16 = np.ones((C,), dtype=np.float32)
  rbd = np.einsum('st,ij,k->sijtk', eye8, eye16,
                  ones16).reshape(P * C, C * P * C) * 0.25
  r8 = np.einsum('st,k->stk', eye8, ones16).reshape(P, P * C)
  x1 = np.einsum('jk,t->jtk', np.eye(2 * H, dtype=np.float32),
                 np.ones((P,), np.float32)).reshape(2 * H, P * 2 * H)
  x2 = np.zeros((C * C, C * P * C), np.float32)
  for i in range(C):
    for k in range(C):
      for t in range(P):
        x2[C * i + k, 128 * i + 16 * t + k] = 1.0
  mask1 = np.kron(eye8, np.ones((H, 2 * H), np.float32))
  mask2 = np.zeros((P * 2 * H, C * P * C), np.float32)
  for s in range(P):
    for i in range(C):
      mask2[32 * s:32 * (s + 1), 128 * i + 16 * s:128 * i + 16 * s + 16] = 1.0
  maskd8 = np.kron(eye8, np.ones((C, C), np.float32))
  a1 = np.tile(eye16, (P, 1))
  a2 = np.tile(eye16, (1, P))
  ms = np.zeros((P * C, P * C), np.float32)
  for sp in range(P):
    for s in range(P):
      for k in range(C):
        ms[16 * sp + k, 16 * s + k] = 1.0 / P
  f = jnp.asarray
  return (f(rbd), f(r8), f(x1), f(x2), f(mask1), f(mask2),
          f(maskd8), f(a1), f(a2), f(ms))


def kernel(node_features, edge_features, edge_sh, edge_index,
           W1, b1, W2, b2, W_byp, bn_w, bn_b):
  ei = edge_index.astype(jnp.int32)
  bf = jnp.bfloat16
  (rbd, r8, x1, x2, mask1, mask2, maskd8, a1, a2, ms) = _np_consts()
  w1bd = (jnp.tile(W1 @ x1, (P, 1)) * mask1).astype(bf)
  b1t = jnp.tile(b1, P).reshape(1, P * 2 * H)
  w2bd = (jnp.tile(W2 @ x2, (P, 1)) * mask2).astype(bf)
  ubd = ((a1 @ b2.reshape(C, C) @ a2) * maskd8 * 0.25).astype(bf)
  rbd = rbd.astype(bf)
  wbd = (a1 @ W_byp @ a2) * maskd8 * 0.25
  bnw_p = jnp.tile(bn_w, P).reshape(1, P * C)
  bnb_p = jnp.tile(bn_b, P).reshape(1, P * C)

  sh_p = edge_sh.reshape(EP, P)
  xdst, cntp, efp = _k1()(ei, node_features, edge_features.T)
  ef_p = efp.reshape(EP, P * H)
  xd_p = xdst.reshape(EP, P * C)
  tp_p = _tc_edge_call(ef_p, xd_p, sh_p, w1bd, b1t, w2bd, ubd, rbd, r8)
  (segp,) = _k3()(tp_p.reshape(NW, EPW, C), ei)
  cnt8 = (cntp[0] + cntp[1])[:N].reshape(NP8, P)
  nf_p = node_features.reshape(NP8, P * C)
  out_p = _tc_final_call(segp.reshape(NC, NP8, P * C), cnt8, nf_p, wbd, r8,
                         ms, bnw_p, bnb_p)
  return out_p.reshape(N, C)


# speedup vs baseline: 1.0041x; 1.0041x over previous
import functools

import jax
import jax.numpy as jnp
import numpy as np
from jax import lax
from jax.experimental import pallas as pl
from jax.experimental.pallas import tpu as pltpu
from jax.experimental.pallas import tpu_sc as plsc

N = 10000
E = 160000
C = 16
H = 16
P = 8
EP = E // P
NC, NS = 2, 16
NW = NC * NS
EPW = E // NW
NPT = 640
RPT = N // NS
ONES_PAD = ((EPW + 15) // 16) * 16


@functools.cache
def _mesh():
  return plsc.VectorSubcoreMesh(
      core_axis_name="c", subcore_axis_name="s", num_cores=NC, num_subcores=NS)


TCH = 1000


def _sc_gather_count(ei_hbm, nf_hbm, eft_hbm, xdst_hbm, cnt_hbm, efp_hbm,
                     idx_v, rows_v, ones_v, zbuf_v, efbuf_v, outbuf_v,
                     cnt_sh, sem):
  cid = lax.axis_index("c")
  sid = lax.axis_index("s")
  wid = sid * NC + cid
  base = wid * EPW
  pltpu.sync_copy(ei_hbm.at[0, pl.ds(base, EPW)], idx_v)
  pltpu.async_copy(nf_hbm.at[idx_v], rows_v, sem).wait()
  pltpu.sync_copy(rows_v, xdst_hbm.at[wid])

  stride16 = lax.iota(jnp.int32, 16) * TCH
  for chunk in range(EPW // TCH):
    for a in range(H):
      pltpu.sync_copy(eft_hbm.at[a, pl.ds(base + chunk * TCH, TCH)],
                      efbuf_v.at[pl.ds(a * TCH, TCH)])

    def trans(e, carry):
      for u in range(16):
        row = plsc.load_gather(efbuf_v, [stride16 + (e * 16 + u)])
        outbuf_v[pl.ds((e * 16 + u) * H, H)] = row
      return carry
    lax.fori_loop(0, TCH // 16, trans, 0)
    pltpu.sync_copy(outbuf_v, efp_hbm.at[wid, pl.ds(chunk * TCH * H, TCH * H)])

  def zfill(i, carry):
    zbuf_v[pl.ds(i * 16, 16)] = jnp.zeros((16,), jnp.float32)
    return carry
  lax.fori_loop(0, NPT // 16, zfill, 0)
  pltpu.sync_copy(zbuf_v, cnt_sh.at[pl.ds(sid * NPT, NPT)])

  def ofill(i, carry):
    ones_v[pl.ds(i * 16, 16)] = jnp.ones((16,), jnp.float32)
    return carry
  lax.fori_loop(0, ONES_PAD // 16, ofill, 0)
  plsc.subcore_barrier()

  pltpu.sync_copy(ei_hbm.at[1, pl.ds(base, EPW)], idx_v)
  pltpu.sync_copy(ones_v.at[pl.ds(0, EPW)], cnt_sh.at[idx_v], add=True)
  plsc.subcore_barrier()
  pltpu.sync_copy(cnt_sh.at[pl.ds(sid * NPT, NPT)],
                  cnt_hbm.at[cid, pl.ds(sid * NPT, NPT)])


@functools.cache
def _k1():
  return pl.kernel(
      _sc_gather_count,
      out_type=[jax.ShapeDtypeStruct((NW, EPW, C), jnp.float32),
                jax.ShapeDtypeStruct((NC, NS * NPT), jnp.float32),
                jax.ShapeDtypeStruct((NW, EPW * H), jnp.float32)],
      mesh=_mesh(),
      scratch_types=[pltpu.VMEM((EPW,), jnp.int32),
                     pltpu.VMEM((EPW, C), jnp.float32),
                     pltpu.VMEM((ONES_PAD,), jnp.float32),
                     pltpu.VMEM((NPT,), jnp.float32),
                     pltpu.VMEM((H * TCH,), jnp.float32),
                     pltpu.VMEM((TCH * H,), jnp.float32),
                     pltpu.VMEM_SHARED((NS * NPT,), jnp.float32),
                     pltpu.SemaphoreType.DMA],
      compiler_params=pltpu.CompilerParams(use_tc_tiling_on_sc=False,
                                           needs_layout_passes=False),
  )


def _sc_scatter(tp_hbm, ei_hbm, seg_hbm, idx_v, rows_v, zrows_v, acc_sh):
  cid = lax.axis_index("c")
  sid = lax.axis_index("s")
  wid = sid * NC + cid

  def zfill(i, carry):
    zrows_v[i, :] = jnp.zeros((16,), jnp.float32)
    return carry
  lax.fori_loop(0, RPT, zfill, 0)
  pltpu.sync_copy(zrows_v, acc_sh.at[pl.ds(sid * RPT, RPT)])
  plsc.subcore_barrier()

  pltpu.sync_copy(ei_hbm.at[1, pl.ds(wid * EPW, EPW)], idx_v)
  pltpu.sync_copy(tp_hbm.at[wid], rows_v)
  pltpu.sync_copy(rows_v, acc_sh.at[idx_v], add=True)
  plsc.subcore_barrier()
  pltpu.sync_copy(acc_sh.at[pl.ds(sid * RPT, RPT)], seg_hbm.at[cid, sid])


@functools.cache
def _k3():
  return pl.kernel(
      _sc_scatter,
      out_type=[jax.ShapeDtypeStruct((NC, NS, RPT, C), jnp.float32)],
      mesh=_mesh(),
      scratch_types=[pltpu.VMEM((EPW,), jnp.int32),
                     pltpu.VMEM((EPW, C), jnp.float32),
                     pltpu.VMEM((RPT, C), jnp.float32),
                     pltpu.VMEM_SHARED((N, C), jnp.float32)],
      compiler_params=pltpu.CompilerParams(use_tc_tiling_on_sc=False),
  )


BLK8 = 1000


def _tc_edge(ef_ref, xd_ref, sh_ref, w1_ref, b1_ref, w2_ref, u_ref,
             r_ref, r8_ref, tp_ref):
  f32 = jnp.float32
  bf = jnp.bfloat16
  xb = xd_ref[...].astype(bf)
  hp = jnp.dot(ef_ref[...].astype(bf), w1_ref[...],
               preferred_element_type=f32) + b1_ref[...]
  h = jnp.maximum(hp, 0.0).astype(bf)
  w = jnp.dot(h, w2_ref[...], preferred_element_type=f32)
  xe = jnp.dot(xb, r_ref[...], preferred_element_type=f32)
  z = w * xe
  parts = [z[:, 128 * i:128 * (i + 1)] for i in range(C)]
  while len(parts) > 1:
    parts = [parts[j] + parts[j + 1] for j in range(0, len(parts), 2)]
  tp = parts[0] + jnp.dot(xb, u_ref[...], preferred_element_type=f32)
  tp_ref[...] = tp * (sh_ref[...] @ r8_ref[...])


def _tc_edge_call(ef, xd, sh, w1bd, b1t, w2bd, ubd, rbd, r8):
  grid = EP // BLK8
  full = lambda shape: pl.BlockSpec(shape, lambda i: (0, 0))
  return pl.pallas_call(
      _tc_edge,
      grid=(grid,),
      in_specs=[
          pl.BlockSpec((BLK8, P * H), lambda i: (i, 0)),
          pl.BlockSpec((BLK8, P * C), lambda i: (i, 0)),
          pl.BlockSpec((BLK8, P), lambda i: (i, 0)),
          full((P * H, P * 2 * H)),
          full((1, P * 2 * H)),
          full((P * 2 * H, C * P * C)),
          full((P * C, P * C)),
          full((P * C, C * P * C)),
          full((P, P * C)),
      ],
      out_specs=pl.BlockSpec((BLK8, P * C), lambda i: (i, 0)),
      out_shape=jax.ShapeDtypeStruct((EP, P * C), jnp.float32),
  )(ef, xd, sh, w1bd, b1t, w2bd, ubd, rbd, r8)


NP8 = N // P


def _tc_final(segp_ref, cnt8_ref, nf_ref, wbd_ref, r8_ref, ms_ref,
              bnw_ref, bnb_ref, out_ref):
  seg = segp_ref[0] + segp_ref[1]
  cnt16 = jnp.maximum(cnt8_ref[...] @ r8_ref[...], 1.0)
  pre = seg / cnt16 + nf_ref[...] @ wbd_ref[...]
  m = jnp.mean(pre, axis=0, keepdims=True)
  msq = jnp.mean(pre * pre, axis=0, keepdims=True)
  mu = m @ ms_ref[...]
  var = msq @ ms_ref[...] - mu * mu
  out_ref[...] = (pre - mu) * lax.rsqrt(var + 1e-5) * bnw_ref[...] + bnb_ref[...]


def _tc_final_call(segp, cnt8, nf_p, wbd, r8, ms, bnw_p, bnb_p):
  full = lambda shape: pl.BlockSpec(shape, lambda: tuple(0 for _ in shape))
  return pl.pallas_call(
      _tc_final,
      in_specs=[
          full((NC, NP8, P * C)),
          full((NP8, P)),
          full((NP8, P * C)),
          full((P * C, P * C)),
          full((P, P * C)),
          full((P * C, P * C)),
          full((1, P * C)),
          full((1, P * C)),
      ],
      out_specs=full((NP8, P * C)),
      out_shape=jax.ShapeDtypeStruct((NP8, P * C), jnp.float32),
  )(segp, cnt8, nf_p, wbd, r8, ms, bnw_p, bnb_p)


def _np_consts():
  eye8 = np.eye(P, dtype=np.float32)
  eye16 = np.eye(C, dtype=np.float32)
  ones16 = np.ones((C,), dtype=np.float32)
  rbd = np.einsum('st,ij,k->sijtk', eye8, eye16,
                  ones16).reshape(P * C, C * P * C) * 0.25
  r8 = np.einsum('st,k->stk', eye8, ones16).reshape(P, P * C)
  x1 = np.einsum('jk,t->jtk', np.eye(2 * H, dtype=np.float32),
                 np.ones((P,), np.float32)).reshape(2 * H, P * 2 * H)
  x2 = np.zeros((C * C, C * P * C), np.float32)
  for i in range(C):
    for k in range(C):
      for t in range(P):
        x2[C * i + k, 128 * i + 16 * t + k] = 1.0
  mask1 = np.kron(eye8, np.ones((H, 2 * H), np.float32))
  mask2 = np.zeros((P * 2 * H, C * P * C), np.float32)
  for s in range(P):
    for i in range(C):
      mask2[32 * s:32 * (s + 1), 128 * i + 16 * s:128 * i + 16 * s + 16] = 1.0
  maskd8 = np.kron(eye8, np.ones((C, C), np.float32))
  a1 = np.tile(eye16, (P, 1))
  a2 = np.tile(eye16, (1, P))
  ms = np.zeros((P * C, P * C), np.float32)
  for sp in range(P):
    for s in range(P):
      for k in range(C):
        ms[16 * sp + k, 16 * s + k] = 1.0 / P
  f = jnp.asarray
  return (f(rbd), f(r8), f(x1), f(x2), f(mask1), f(mask2),
          f(maskd8), f(a1), f(a2), f(ms))


def kernel(node_features, edge_features, edge_sh, edge_index,
           W1, b1, W2, b2, W_byp, bn_w, bn_b):
  ei = edge_index.astype(jnp.int32)
  bf = jnp.bfloat16
  (rbd, r8, x1, x2, mask1, mask2, maskd8, a1, a2, ms) = _np_consts()
  w1bd = (jnp.tile(W1 @ x1, (P, 1)) * mask1).astype(bf)
  b1t = jnp.tile(b1, P).reshape(1, P * 2 * H)
  w2bd = (jnp.tile(W2 @ x2, (P, 1)) * mask2).astype(bf)
  ubd = ((a1 @ b2.reshape(C, C) @ a2) * maskd8 * 0.25).astype(bf)
  rbd = rbd.astype(bf)
  wbd = (a1 @ W_byp @ a2) * maskd8 * 0.25
  bnw_p = jnp.tile(bn_w, P).reshape(1, P * C)
  bnb_p = jnp.tile(bn_b, P).reshape(1, P * C)

  sh_p = edge_sh.reshape(EP, P)
  xdst, cntp, efp = _k1()(ei, node_features, edge_features.T)
  ef_p = efp.reshape(EP, P * H)
  xd_p = xdst.reshape(EP, P * C)
  tp_p = _tc_edge_call(ef_p, xd_p, sh_p, w1bd, b1t, w2bd, ubd, rbd, r8)
  (segp,) = _k3()(tp_p.reshape(NW, EPW, C), ei)
  cnt8 = (cntp[0] + cntp[1])[:N].reshape(NP8, P)
  nf_p = node_features.reshape(NP8, P * C)
  out_p = _tc_final_call(segp.reshape(NC, NP8, P * C), cnt8, nf_p, wbd, r8,
                         ms, bnw_p, bnb_p)
  return out_p.reshape(N, C)


# speedup vs baseline: 1.1620x; 1.1573x over previous
import functools

import jax
import jax.numpy as jnp
import numpy as np
from jax import lax
from jax.experimental import pallas as pl
from jax.experimental.pallas import tpu as pltpu
from jax.experimental.pallas import tpu_sc as plsc

N = 10000
E = 160000
C = 16
H = 16
P = 8
EP = E // P
NC, NS = 2, 16
NW = NC * NS
EPW = E // NW
NPT = 640
RPT = N // NS
ONES_PAD = ((EPW + 15) // 16) * 16


@functools.cache
def _mesh():
  return plsc.VectorSubcoreMesh(
      core_axis_name="c", subcore_axis_name="s", num_cores=NC, num_subcores=NS)


def _sc_gather_count(ei_hbm, nf_hbm, xdst_hbm, cnt_hbm,
                     idx_v, rows_v, ones_v, zbuf_v, cnt_sh, sem):
  cid = lax.axis_index("c")
  sid = lax.axis_index("s")
  wid = sid * NC + cid
  base = wid * EPW
  pltpu.sync_copy(ei_hbm.at[0, pl.ds(base, EPW)], idx_v)
  pltpu.async_copy(nf_hbm.at[idx_v], rows_v, sem).wait()
  pltpu.sync_copy(rows_v, xdst_hbm.at[wid])

  def zfill(i, carry):
    zbuf_v[pl.ds(i * 16, 16)] = jnp.zeros((16,), jnp.float32)
    return carry
  lax.fori_loop(0, NPT // 16, zfill, 0)
  pltpu.sync_copy(zbuf_v, cnt_sh.at[pl.ds(sid * NPT, NPT)])

  def ofill(i, carry):
    ones_v[pl.ds(i * 16, 16)] = jnp.ones((16,), jnp.float32)
    return carry
  lax.fori_loop(0, ONES_PAD // 16, ofill, 0)
  plsc.subcore_barrier()

  pltpu.sync_copy(ei_hbm.at[1, pl.ds(base, EPW)], idx_v)
  pltpu.sync_copy(ones_v.at[pl.ds(0, EPW)], cnt_sh.at[idx_v], add=True)
  plsc.subcore_barrier()
  pltpu.sync_copy(cnt_sh.at[pl.ds(sid * NPT, NPT)],
                  cnt_hbm.at[cid, pl.ds(sid * NPT, NPT)])


@functools.cache
def _k1():
  return pl.kernel(
      _sc_gather_count,
      out_type=[jax.ShapeDtypeStruct((NW, EPW, C), jnp.float32),
                jax.ShapeDtypeStruct((NC, NS * NPT), jnp.float32)],
      mesh=_mesh(),
      scratch_types=[pltpu.VMEM((EPW,), jnp.int32),
                     pltpu.VMEM((EPW, C), jnp.float32),
                     pltpu.VMEM((ONES_PAD,), jnp.float32),
                     pltpu.VMEM((NPT,), jnp.float32),
                     pltpu.VMEM_SHARED((NS * NPT,), jnp.float32),
                     pltpu.SemaphoreType.DMA],
      compiler_params=pltpu.CompilerParams(use_tc_tiling_on_sc=False),
  )


def _sc_scatter(tp_hbm, ei_hbm, seg_hbm, idx_v, rows_v, zrows_v, acc_sh):
  cid = lax.axis_index("c")
  sid = lax.axis_index("s")
  wid = sid * NC + cid

  def zfill(i, carry):
    zrows_v[i, :] = jnp.zeros((16,), jnp.float32)
    return carry
  lax.fori_loop(0, RPT, zfill, 0)
  pltpu.sync_copy(zrows_v, acc_sh.at[pl.ds(sid * RPT, RPT)])
  plsc.subcore_barrier()

  pltpu.sync_copy(ei_hbm.at[1, pl.ds(wid * EPW, EPW)], idx_v)
  pltpu.sync_copy(tp_hbm.at[wid], rows_v)
  pltpu.sync_copy(rows_v, acc_sh.at[idx_v], add=True)
  plsc.subcore_barrier()
  pltpu.sync_copy(acc_sh.at[pl.ds(sid * RPT, RPT)], seg_hbm.at[cid, sid])


@functools.cache
def _k3():
  return pl.kernel(
      _sc_scatter,
      out_type=[jax.ShapeDtypeStruct((NC, NS, RPT, C), jnp.float32)],
      mesh=_mesh(),
      scratch_types=[pltpu.VMEM((EPW,), jnp.int32),
                     pltpu.VMEM((EPW, C), jnp.float32),
                     pltpu.VMEM((RPT, C), jnp.float32),
                     pltpu.VMEM_SHARED((N, C), jnp.float32)],
      compiler_params=pltpu.CompilerParams(use_tc_tiling_on_sc=False),
  )


BLK8 = 1000


def _tc_edge(ef_ref, xd_ref, sh_ref, w1_ref, b1_ref, w2_ref, u_ref,
             r_ref, r8_ref, tp_ref):
  f32 = jnp.float32
  bf = jnp.bfloat16
  xb = xd_ref[...].astype(bf)
  hp = jnp.dot(ef_ref[...].astype(bf), w1_ref[...],
               preferred_element_type=f32) + b1_ref[...]
  h = jnp.maximum(hp, 0.0).astype(bf)
  w = jnp.dot(h, w2_ref[...], preferred_element_type=f32)
  xe = jnp.dot(xb, r_ref[...], preferred_element_type=f32)
  z = w * xe
  parts = [z[:, 128 * i:128 * (i + 1)] for i in range(C)]
  while len(parts) > 1:
    parts = [parts[j] + parts[j + 1] for j in range(0, len(parts), 2)]
  tp = parts[0] + jnp.dot(xb, u_ref[...], preferred_element_type=f32)
  tp_ref[...] = tp * (sh_ref[...] @ r8_ref[...])


def _tc_edge_call(ef, xd, sh, w1bd, b1t, w2bd, ubd, rbd, r8):
  grid = EP // BLK8
  full = lambda shape: pl.BlockSpec(shape, lambda i: (0, 0))
  return pl.pallas_call(
      _tc_edge,
      grid=(grid,),
      in_specs=[
          pl.BlockSpec((BLK8, P * H), lambda i: (i, 0)),
          pl.BlockSpec((BLK8, P * C), lambda i: (i, 0)),
          pl.BlockSpec((BLK8, P), lambda i: (i, 0)),
          full((P * H, P * 2 * H)),
          full((1, P * 2 * H)),
          full((P * 2 * H, C * P * C)),
          full((P * C, P * C)),
          full((P * C, C * P * C)),
          full((P, P * C)),
      ],
      out_specs=pl.BlockSpec((BLK8, P * C), lambda i: (i, 0)),
      out_shape=jax.ShapeDtypeStruct((EP, P * C), jnp.float32),
  )(ef, xd, sh, w1bd, b1t, w2bd, ubd, rbd, r8)


NP8 = N // P


def _tc_final(segp_ref, cnt8_ref, nf_ref, wbd_ref, r8_ref, ms_ref,
              bnw_ref, bnb_ref, out_ref):
  seg = segp_ref[0] + segp_ref[1]
  cnt16 = jnp.maximum(cnt8_ref[...] @ r8_ref[...], 1.0)
  pre = seg / cnt16 + nf_ref[...] @ wbd_ref[...]
  m = jnp.mean(pre, axis=0, keepdims=True)
  msq = jnp.mean(pre * pre, axis=0, keepdims=True)
  mu = m @ ms_ref[...]
  var = msq @ ms_ref[...] - mu * mu
  out_ref[...] = (pre - mu) * lax.rsqrt(var + 1e-5) * bnw_ref[...] + bnb_ref[...]


def _tc_final_call(segp, cnt8, nf_p, wbd, r8, ms, bnw_p, bnb_p):
  full = lambda shape: pl.BlockSpec(shape, lambda: tuple(0 for _ in shape))
  return pl.pallas_call(
      _tc_final,
      in_specs=[
          full((NC, NP8, P * C)),
          full((NP8, P)),
          full((NP8, P * C)),
          full((P * C, P * C)),
          full((P, P * C)),
          full((P * C, P * C)),
          full((1, P * C)),
          full((1, P * C)),
      ],
      out_specs=full((NP8, P * C)),
      out_shape=jax.ShapeDtypeStruct((NP8, P * C), jnp.float32),
  )(segp, cnt8, nf_p, wbd, r8, ms, bnw_p, bnb_p)


def _np_consts():
  eye8 = np.eye(P, dtype=np.float32)
  eye16 = np.eye(C, dtype=np.float32)
  ones16 = np.ones((C,), dtype=np.float32)
  rbd = np.einsum('st,ij,k->sijtk', eye8, eye16,
                  ones16).reshape(P * C, C * P * C) * 0.25
  r8 = np.einsum('st,k->stk', eye8, ones16).reshape(P, P * C)
  x1 = np.einsum('jk,t->jtk', np.eye(2 * H, dtype=np.float32),
                 np.ones((P,), np.float32)).reshape(2 * H, P * 2 * H)
  x2 = np.zeros((C * C, C * P * C), np.float32)
  for i in range(C):
    for k in range(C):
      for t in range(P):
        x2[C * i + k, 128 * i + 16 * t + k] = 1.0
  mask1 = np.kron(eye8, np.ones((H, 2 * H), np.float32))
  mask2 = np.zeros((P * 2 * H, C * P * C), np.float32)
  for s in range(P):
    for i in range(C):
      mask2[32 * s:32 * (s + 1), 128 * i + 16 * s:128 * i + 16 * s + 16] = 1.0
  maskd8 = np.kron(eye8, np.ones((C, C), np.float32))
  a1 = np.tile(eye16, (P, 1))
  a2 = np.tile(eye16, (1, P))
  ms = np.zeros((P * C, P * C), np.float32)
  for sp in range(P):
    for s in range(P):
      for k in range(C):
        ms[16 * sp + k, 16 * s + k] = 1.0 / P
  f = jnp.asarray
  return (f(rbd), f(r8), f(x1), f(x2), f(mask1), f(mask2),
          f(maskd8), f(a1), f(a2), f(ms))


def kernel(node_features, edge_features, edge_sh, edge_index,
           W1, b1, W2, b2, W_byp, bn_w, bn_b):
  ei = edge_index.astype(jnp.int32)
  bf = jnp.bfloat16
  (rbd, r8, x1, x2, mask1, mask2, maskd8, a1, a2, ms) = _np_consts()
  w1bd = (jnp.tile(W1 @ x1, (P, 1)) * mask1).astype(bf)
  b1t = jnp.tile(b1, P).reshape(1, P * 2 * H)
  w2bd = (jnp.tile(W2 @ x2, (P, 1)) * mask2).astype(bf)
  ubd = ((a1 @ b2.reshape(C, C) @ a2) * maskd8 * 0.25).astype(bf)
  rbd = rbd.astype(bf)
  wbd = (a1 @ W_byp @ a2) * maskd8 * 0.25
  bnw_p = jnp.tile(bn_w, P).reshape(1, P * C)
  bnb_p = jnp.tile(bn_b, P).reshape(1, P * C)

  ef_p = edge_features.reshape(EP, P * H)
  sh_p = edge_sh.reshape(EP, P)
  xdst, cntp = _k1()(ei, node_features)
  xd_p = xdst.reshape(EP, P * C)
  tp_p = _tc_edge_call(ef_p, xd_p, sh_p, w1bd, b1t, w2bd, ubd, rbd, r8)
  (segp,) = _k3()(tp_p.reshape(NW, EPW, C), ei)
  cnt8 = (cntp[0] + cntp[1])[:N].reshape(NP8, P)
  nf_p = node_features.reshape(NP8, P * C)
  out_p = _tc_final_call(segp.reshape(NC, NP8, P * C), cnt8, nf_p, wbd, r8,
                         ms, bnw_p, bnb_p)
  return out_p.reshape(N, C)
